# baseline (speedup 1.0000x reference)
import functools

import jax
import jax.numpy as jnp
from jax import lax
from jax.experimental import pallas as pl
from jax.experimental.pallas import tpu as pltpu
from jax.experimental.pallas import tpu_sc as plsc

N = 10000
D = 128
H = 16
E = 320000
IBLK = 128
NC = 2
NS = 16
NW = NC * NS
BPT = 80
EPAD = NW * BPT * IBLK
EBP = EPAD // IBLK
PH = 20
NPHASE = BPT // PH
N_ACC = 10016
ZROWS = N_ACC // NS
ROWS_PER_TILE = N // NS

F32 = jnp.float32


def _sc_mesh():
    return plsc.VectorSubcoreMesh(
        core_axis_name="c", subcore_axis_name="s", num_cores=NC, num_subcores=NS)


DEG_BPT = EBP // NS
DPH = 10


def _sc_deg_conv1(xw1, src2, dst2, zeros):
    def body(xw_hbm, src_hbm, dst_hbm, z_hbm, out_hbm, y1_hbm, dinv_hbm,
             acc_sh, y_sh, sidx_all, didx_all, degidx, ones_v,
             rows0, rows1, xbuf, dbuf):
        c = lax.axis_index("c")
        s = lax.axis_index("s")
        wid = c * NS + s
        r0 = s * ROWS_PER_TILE

        def stage(t1, t3, t4):
            d1 = pltpu.async_copy(z_hbm.at[pl.ds(s * ZROWS, ZROWS)],
                                  acc_sh.at[pl.ds(s * ZROWS, ZROWS)], t1)
            d3 = pltpu.async_copy(src_hbm.at[pl.ds(wid * BPT, BPT)], sidx_all, t3)
            d4 = pltpu.async_copy(xw_hbm.at[pl.ds(r0, ROWS_PER_TILE)], xbuf, t4)
            pltpu.sync_copy(dst_hbm.at[pl.ds(wid * BPT, BPT)], didx_all)
            d1.wait(); d3.wait(); d4.wait()

        pl.run_scoped(stage, pltpu.SemaphoreType.DMA,
                      pltpu.SemaphoreType.DMA, pltpu.SemaphoreType.DMA)

        def fill(i, carry):
            ones_v[i, :] = jnp.ones((H,), F32)
            return carry
        lax.fori_loop(0, IBLK, fill, 0)
        plsc.subcore_barrier()

        def deg_run(ssem):
            for r in range(2):
                pltpu.sync_copy(
                    dst_hbm.at[pl.ds(s * DEG_BPT + r * (DEG_BPT // 2),
                                     DEG_BPT // 2)], degidx)
                ngrp = (DEG_BPT // 2) // DPH
                for g in range(ngrp):
                    def launch(j, carry):
                        pltpu.async_copy(ones_v, acc_sh.at[degidx.at[g * DPH + j]],
                                         ssem, add=True)
                        return carry
                    lax.fori_loop(0, DPH, launch, 0)
                    if g > 0:
                        pltpu.make_async_copy(z_hbm.at[pl.ds(0, DPH * IBLK)],
                                              rows0, ssem).wait()
                pltpu.make_async_copy(z_hbm.at[pl.ds(0, DPH * IBLK)],
                                      rows0, ssem).wait()

        pl.run_scoped(deg_run, pltpu.SemaphoreType.DMA)
        plsc.subcore_barrier()

        pltpu.sync_copy(acc_sh.at[pl.ds(r0, ROWS_PER_TILE)], dbuf)
        plsc.subcore_barrier()
        pltpu.sync_copy(z_hbm.at[pl.ds(s * ZROWS, ZROWS)],
                        acc_sh.at[pl.ds(s * ZROWS, ZROWS)])

        magic = jnp.full((H,), 0x5F3759DF, jnp.int32)

        def rsqrt_row(i, carry):
            xv = dbuf[i, :] + 1.0
            bits = magic - lax.shift_right_logical(plsc.bitcast(xv, jnp.int32), 1)
            yv = plsc.bitcast(bits, F32)
            for _ in range(3):
                yv = yv * (1.5 - 0.5 * xv * yv * yv)
            dbuf[i, :] = yv
            xbuf[i, :] = xbuf[i, :] * yv
            return carry
        lax.fori_loop(0, ROWS_PER_TILE, rsqrt_row, 0)

        pltpu.sync_copy(xbuf, y_sh.at[pl.ds(r0, ROWS_PER_TILE)])

        @pl.when(c == 0)
        def _():
            pltpu.sync_copy(xbuf, y1_hbm.at[pl.ds(r0, ROWS_PER_TILE)])
            pltpu.sync_copy(dbuf, dinv_hbm.at[pl.ds(r0, ROWS_PER_TILE)])
        plsc.subcore_barrier()

        rows = (rows0, rows1)

        def fire_gathers(p, buf, gsem):
            def launch(j, carry):
                pltpu.async_copy(y_sh.at[sidx_all.at[p * DPH + j]],
                                 buf.at[pl.ds(j * IBLK, IBLK)], gsem)
                return carry
            lax.fori_loop(0, DPH, launch, 0)

        def fire_scatters(p, buf, ssem):
            def launch(j, carry):
                pltpu.async_copy(buf.at[pl.ds(j * IBLK, IBLK)],
                                 acc_sh.at[didx_all.at[p * DPH + j]],
                                 ssem, add=True)
                return carry
            lax.fori_loop(0, DPH, launch, 0)

        def drain(sem):
            pltpu.make_async_copy(z_hbm.at[pl.ds(0, DPH * IBLK)],
                                  rows0, sem).wait()

        nphase = BPT // DPH

        def run(gsem, ssem):
            fire_gathers(0, rows[0], gsem)
            for p in range(nphase):
                cur = rows[p % 2]
                drain(gsem)
                fire_scatters(p, cur, ssem)
                if p + 1 < nphase:
                    fire_gathers(p + 1, rows[(p + 1) % 2], gsem)
                drain(ssem)

        pl.run_scoped(run, pltpu.SemaphoreType.DMA, pltpu.SemaphoreType.DMA)
        plsc.subcore_barrier()
        pltpu.sync_copy(acc_sh.at[pl.ds(r0, ROWS_PER_TILE)],
                        out_hbm.at[c].at[pl.ds(r0, ROWS_PER_TILE)])

    f = pl.kernel(
        body,
        out_type=(jax.ShapeDtypeStruct((NC, N, H), F32),
                  jax.ShapeDtypeStruct((N, H), F32),
                  jax.ShapeDtypeStruct((N, H), F32)),
        mesh=_sc_mesh(),
        compiler_params=pltpu.CompilerParams(use_tc_tiling_on_sc=False,
                                             needs_layout_passes=False),
        scratch_types=[
            pltpu.VMEM_SHARED((N_ACC, H), F32),
            pltpu.VMEM_SHARED((N, H), F32),
            pltpu.VMEM((BPT, IBLK), jnp.int32),
            pltpu.VMEM((BPT, IBLK), jnp.int32),
            pltpu.VMEM((DEG_BPT // 2, IBLK), jnp.int32),
            pltpu.VMEM((IBLK, H), F32),
            pltpu.VMEM((DPH * IBLK, H), F32),
            pltpu.VMEM((DPH * IBLK, H), F32),
            pltpu.VMEM((ROWS_PER_TILE, H), F32),
            pltpu.VMEM((ROWS_PER_TILE, H), F32),
        ],
    )
    return f(xw1, src2, dst2, zeros)


def _sc_gather_scatter(y, src2, dst2, zeros):
    def body(y_hbm, src_hbm, dst_hbm, z_hbm, out_hbm,
             acc_sh, y_sh, sidx_all, didx_all, rows0, rows1):
        c = lax.axis_index("c")
        s = lax.axis_index("s")
        wid = c * NS + s
        def stage(t1, t2, t3, t4):
            d1 = pltpu.async_copy(z_hbm.at[pl.ds(s * ZROWS, ZROWS)],
                                  acc_sh.at[pl.ds(s * ZROWS, ZROWS)], t1)
            d2 = pltpu.async_copy(y_hbm.at[pl.ds(s * ROWS_PER_TILE, ROWS_PER_TILE)],
                                  y_sh.at[pl.ds(s * ROWS_PER_TILE, ROWS_PER_TILE)], t2)
            d3 = pltpu.async_copy(src_hbm.at[pl.ds(wid * BPT, BPT)], sidx_all, t3)
            d4 = pltpu.async_copy(dst_hbm.at[pl.ds(wid * BPT, BPT)], didx_all, t4)
            d1.wait(); d2.wait(); d3.wait(); d4.wait()

        pl.run_scoped(stage, pltpu.SemaphoreType.DMA, pltpu.SemaphoreType.DMA,
                      pltpu.SemaphoreType.DMA, pltpu.SemaphoreType.DMA)
        plsc.subcore_barrier()

        rows = (rows0, rows1)

        def fire_gathers(p, buf, gsem):
            def launch(j, carry):
                pltpu.async_copy(y_sh.at[sidx_all.at[p * PH + j]],
                                 buf.at[pl.ds(j * IBLK, IBLK)], gsem)
                return carry
            lax.fori_loop(0, PH, launch, 0)

        def fire_scatters(p, buf, ssem):
            def launch(j, carry):
                pltpu.async_copy(buf.at[pl.ds(j * IBLK, IBLK)],
                                 acc_sh.at[didx_all.at[p * PH + j]],
                                 ssem, add=True)
                return carry
            lax.fori_loop(0, PH, launch, 0)

        def drain(sem):
            pltpu.make_async_copy(z_hbm.at[pl.ds(0, PH * IBLK)],
                                  rows0, sem).wait()

        def run(gsem, ssem):
            fire_gathers(0, rows[0], gsem)
            for p in range(NPHASE):
                cur = rows[p % 2]
                drain(gsem)
                fire_scatters(p, cur, ssem)
                if p + 1 < NPHASE:
                    fire_gathers(p + 1, rows[(p + 1) % 2], gsem)
                drain(ssem)

        pl.run_scoped(run, pltpu.SemaphoreType.DMA, pltpu.SemaphoreType.DMA)
        plsc.subcore_barrier()
        pltpu.sync_copy(acc_sh.at[pl.ds(s * ROWS_PER_TILE, ROWS_PER_TILE)],
                        out_hbm.at[c].at[pl.ds(s * ROWS_PER_TILE, ROWS_PER_TILE)])

    f = pl.kernel(
        body,
        out_type=jax.ShapeDtypeStruct((NC, N, H), F32),
        mesh=_sc_mesh(),
        compiler_params=pltpu.CompilerParams(use_tc_tiling_on_sc=False),
        scratch_types=[
            pltpu.VMEM_SHARED((N_ACC, H), F32),
            pltpu.VMEM_SHARED((N, H), F32),
            pltpu.VMEM((BPT, IBLK), jnp.int32),
            pltpu.VMEM((BPT, IBLK), jnp.int32),
            pltpu.VMEM((PH * IBLK, H), F32),
            pltpu.VMEM((PH * IBLK, H), F32),
        ],
    )
    return f(y, src2, dst2, zeros)


NS8 = N // 8


def _tc_xw(x, W1):
    def body(x_ref, w_ref, o_ref):
        o_ref[...] = jnp.dot(x_ref[...], w_ref[...], preferred_element_type=F32)

    return pl.pallas_call(
        body, out_shape=jax.ShapeDtypeStruct((N, H), F32),
    )(x, W1)


def _tc_combine_mm(s0, s1, y, dinv, b, Wbd):
    def body(s0_ref, s1_ref, y_ref, dinv_ref, b_ref, w_ref, o_ref):
        dinv = dinv_ref[...]
        h = jnp.maximum(dinv * (s0_ref[...] + s1_ref[...] + y_ref[...]) + b_ref[...], 0.0)
        o_ref[...] = jnp.dot(h, w_ref[...], preferred_element_type=F32) * dinv

    return pl.pallas_call(
        body, out_shape=jax.ShapeDtypeStruct((NS8, 128), F32),
    )(s0, s1, y, dinv, b, Wbd)


def _tc_combine(s0, s1, y, dinv, b):
    def body(s0_ref, s1_ref, y_ref, dinv_ref, b_ref, o_ref):
        o_ref[...] = jnp.maximum(
            dinv_ref[...] * (s0_ref[...] + s1_ref[...] + y_ref[...]) + b_ref[...], 0.0)

    return pl.pallas_call(
        body, out_shape=jax.ShapeDtypeStruct((NS8, 128), F32),
    )(s0, s1, y, dinv, b)


MV_BK = 6400


def _tc_matvec_heads(flat, Wp1, Wv1, Wp2t, bp1, bp2, bv1, wiv, biv, wev, bev):
    grid = (N * H) // MV_BK
    wrows = N // grid

    def body(f_ref, a_ref, b_ref, wp2_ref, bp1_ref, bp2_ref, bv1_ref,
             wiv_ref, biv_ref, wev_ref, bev_ref,
             x_ref, ev_ref, iv_ref, pacc, vacc, wp2_full):
        k = pl.program_id(0)

        @pl.when(k == 0)
        def _():
            pacc[...] = jnp.zeros_like(pacc)
            vacc[...] = jnp.zeros_like(vacc)

        f = f_ref[...]
        pacc[...] += jnp.dot(f, a_ref[...], preferred_element_type=F32)
        vacc[...] += jnp.dot(f, b_ref[...], preferred_element_type=F32)
        wp2_full[pl.ds(k * wrows, wrows), :] = wp2_ref[...]

        @pl.when(k == grid - 1)
        def _():
            ph = jnp.maximum(pacc[...] + bp1_ref[...], 0.0)
            x_ref[...] = lax.dot_general(
                ph, wp2_full[...], (((1,), (1,)), ((), ())),
                preferred_element_type=F32) + bp2_ref[...]
            V = jnp.maximum(vacc[...] + bv1_ref[...], 0.0)
            iv_ref[...] = jnp.sum(V * wiv_ref[...], axis=1, keepdims=True) + biv_ref[...]
            ev_ref[...] = jnp.sum(V * wev_ref[...], axis=1, keepdims=True) + bev_ref[...]

    c0 = lambda k: (0, 0)
    return pl.pallas_call(
        body,
        grid=(grid,),
        in_specs=[
            pl.BlockSpec((1, MV_BK), lambda k: (0, k)),
            pl.BlockSpec((MV_BK, 256), lambda k: (k, 0)),
            pl.BlockSpec((MV_BK, 256), lambda k: (k, 0)),
            pl.BlockSpec((wrows, 256), lambda k: (k, 0)),
            pl.BlockSpec((1, 256), c0),
            pl.BlockSpec((1, N), c0),
            pl.BlockSpec((1, 256), c0),
            pl.BlockSpec((1, 256), c0),
            pl.BlockSpec((1, 1), c0),
            pl.BlockSpec((1, 256), c0),
            pl.BlockSpec((1, 1), c0),
        ],
        out_specs=(pl.BlockSpec((1, N), c0),
                   pl.BlockSpec((1, 1), c0),
                   pl.BlockSpec((1, 1), c0)),
        out_shape=(jax.ShapeDtypeStruct((1, N), F32),
                   jax.ShapeDtypeStruct((1, 1), F32),
                   jax.ShapeDtypeStruct((1, 1), F32)),
        scratch_shapes=[
            pltpu.VMEM((1, 256), F32),
            pltpu.VMEM((1, 256), F32),
            pltpu.VMEM((N, 256), F32),
        ],
        compiler_params=pltpu.CompilerParams(vmem_limit_bytes=63 * 1024 * 1024),
    )(flat, Wp1, Wv1, Wp2t, bp1, bp2, bv1, wiv, biv, wev, bev)


def kernel(x, edge_index, W1, b1, W2, b2, W3, b3, Wp1, bp1, Wp2, bp2,
           Wv1, bv1, Wiv, biv, Wev, bev):
    ei = edge_index.astype(jnp.int32)
    pad = EPAD - E
    src2 = jnp.concatenate([ei[0], jnp.zeros((pad,), jnp.int32)]).reshape(EBP, IBLK)
    dpad = N + (jnp.arange(pad, dtype=jnp.int32) % (N_ACC - N))
    dst2 = jnp.concatenate([ei[1], dpad]).reshape(EBP, IBLK)
    zeros = jnp.zeros((N_ACC, H), F32)

    eye8 = jnp.eye(8, dtype=F32)

    def tile8(b):
        return jnp.tile(b.reshape(1, H), (1, 8))

    xw1 = _tc_xw(x, W1)
    p1, y1, dinv = _sc_deg_conv1(xw1, src2, dst2, zeros)
    y1s = y1.reshape(NS8, 128)
    dinvs = dinv.reshape(NS8, 128)
    y2s = _tc_combine_mm(p1[0].reshape(NS8, 128), p1[1].reshape(NS8, 128),
                         y1s, dinvs, tile8(b1), jnp.kron(eye8, W2))

    p2 = _sc_gather_scatter(y2s.reshape(N, H), src2, dst2, zeros)
    y3s = _tc_combine_mm(p2[0].reshape(NS8, 128), p2[1].reshape(NS8, 128),
                         y2s, dinvs, tile8(b2), jnp.kron(eye8, W3))

    p3 = _sc_gather_scatter(y3s.reshape(N, H), src2, dst2, zeros)
    h3s = _tc_combine(p3[0].reshape(NS8, 128), p3[1].reshape(NS8, 128),
                      y3s, dinvs, tile8(b3))

    flat = h3s.reshape(1, N * H)
    X, eV, iV = _tc_matvec_heads(
        flat, Wp1, Wv1, Wp2.T, bp1.reshape(1, 256), bp2.reshape(1, N),
        bv1.reshape(1, 256), Wiv.reshape(1, 256), biv.reshape(1, 1),
        Wev.reshape(1, 256), bev.reshape(1, 1))
    return (X, eV, iV)


# speedup vs baseline: 1.0099x; 1.0099x over previous
import functools

import jax
import jax.numpy as jnp
from jax import lax
from jax.experimental import pallas as pl
from jax.experimental.pallas import tpu as pltpu
from jax.experimental.pallas import tpu_sc as plsc

N = 10000
D = 128
H = 16
E = 320000
IBLK = 128
NC = 2
NS = 16
NW = NC * NS
BPT = 80
EPAD = NW * BPT * IBLK
EBP = EPAD // IBLK
PH = 20
NPHASE = BPT // PH
N_ACC = 10016
ZROWS = N_ACC // NS
ROWS_PER_TILE = N // NS

F32 = jnp.float32


def _sc_mesh():
    return plsc.VectorSubcoreMesh(
        core_axis_name="c", subcore_axis_name="s", num_cores=NC, num_subcores=NS)


def _sc_degree(dst2, zeros):
    def body(dst_hbm, z_hbm, out_hbm, acc_sh, didx_all, ones_v, drows):
        c = lax.axis_index("c")
        s = lax.axis_index("s")
        wid = c * NS + s
        pltpu.sync_copy(z_hbm.at[pl.ds(s * ZROWS, ZROWS)],
                        acc_sh.at[pl.ds(s * ZROWS, ZROWS)])
        pltpu.sync_copy(dst_hbm.at[pl.ds(wid * BPT, BPT)], didx_all)

        def fill(i, carry):
            ones_v[i, :] = jnp.ones((H,), F32)
            return carry
        lax.fori_loop(0, IBLK, fill, 0)
        plsc.subcore_barrier()

        def fire(g, j, ssem):
            pltpu.async_copy(ones_v, acc_sh.at[didx_all.at[g * PH + j]],
                             ssem, add=True)

        def run(ssem):
            for g in range(NPHASE):
                def launch(j, carry):
                    fire(g, j, ssem)
                    return carry
                lax.fori_loop(0, PH, launch, 0)
                if g > 0:
                    pltpu.make_async_copy(z_hbm.at[pl.ds(0, PH * IBLK)],
                                          drows, ssem).wait()
            pltpu.make_async_copy(z_hbm.at[pl.ds(0, PH * IBLK)],
                                  drows, ssem).wait()

        pl.run_scoped(run, pltpu.SemaphoreType.DMA)
        plsc.subcore_barrier()
        pltpu.sync_copy(acc_sh.at[pl.ds(s * ROWS_PER_TILE, ROWS_PER_TILE)],
                        out_hbm.at[c].at[pl.ds(s * ROWS_PER_TILE, ROWS_PER_TILE)])

    f = pl.kernel(
        body,
        out_type=jax.ShapeDtypeStruct((NC, N, H), F32),
        mesh=_sc_mesh(),
        compiler_params=pltpu.CompilerParams(use_tc_tiling_on_sc=False),
        scratch_types=[
            pltpu.VMEM_SHARED((N_ACC, H), F32),
            pltpu.VMEM((BPT, IBLK), jnp.int32),
            pltpu.VMEM((IBLK, H), F32),
            pltpu.VMEM((PH * IBLK, H), F32),
        ],
    )
    return f(dst2, zeros)


def _sc_gather_scatter(y, src2, dst2, zeros):
    def body(y_hbm, src_hbm, dst_hbm, z_hbm, out_hbm,
             acc_sh, y_sh, sidx_all, didx_all, rows0, rows1):
        c = lax.axis_index("c")
        s = lax.axis_index("s")
        wid = c * NS + s
        def stage(t1, t2, t3, t4):
            d1 = pltpu.async_copy(z_hbm.at[pl.ds(s * ZROWS, ZROWS)],
                                  acc_sh.at[pl.ds(s * ZROWS, ZROWS)], t1)
            d2 = pltpu.async_copy(y_hbm.at[pl.ds(s * ROWS_PER_TILE, ROWS_PER_TILE)],
                                  y_sh.at[pl.ds(s * ROWS_PER_TILE, ROWS_PER_TILE)], t2)
            d3 = pltpu.async_copy(src_hbm.at[pl.ds(wid * BPT, BPT)], sidx_all, t3)
            d4 = pltpu.async_copy(dst_hbm.at[pl.ds(wid * BPT, BPT)], didx_all, t4)
            d1.wait(); d2.wait(); d3.wait(); d4.wait()

        pl.run_scoped(stage, pltpu.SemaphoreType.DMA, pltpu.SemaphoreType.DMA,
                      pltpu.SemaphoreType.DMA, pltpu.SemaphoreType.DMA)
        plsc.subcore_barrier()

        rows = (rows0, rows1)

        def fire_gathers(p, buf, gsem):
            def launch(j, carry):
                pltpu.async_copy(y_sh.at[sidx_all.at[p * PH + j]],
                                 buf.at[pl.ds(j * IBLK, IBLK)], gsem)
                return carry
            lax.fori_loop(0, PH, launch, 0)

        def fire_scatters(p, buf, ssem):
            def launch(j, carry):
                pltpu.async_copy(buf.at[pl.ds(j * IBLK, IBLK)],
                                 acc_sh.at[didx_all.at[p * PH + j]],
                                 ssem, add=True)
                return carry
            lax.fori_loop(0, PH, launch, 0)

        def drain(sem):
            pltpu.make_async_copy(z_hbm.at[pl.ds(0, PH * IBLK)],
                                  rows0, sem).wait()

        def run(gsem, ssem):
            fire_gathers(0, rows[0], gsem)
            for p in range(NPHASE):
                cur = rows[p % 2]
                drain(gsem)
                fire_scatters(p, cur, ssem)
                if p + 1 < NPHASE:
                    fire_gathers(p + 1, rows[(p + 1) % 2], gsem)
                drain(ssem)

        pl.run_scoped(run, pltpu.SemaphoreType.DMA, pltpu.SemaphoreType.DMA)
        plsc.subcore_barrier()
        pltpu.sync_copy(acc_sh.at[pl.ds(s * ROWS_PER_TILE, ROWS_PER_TILE)],
                        out_hbm.at[c].at[pl.ds(s * ROWS_PER_TILE, ROWS_PER_TILE)])

    f = pl.kernel(
        body,
        out_type=jax.ShapeDtypeStruct((NC, N, H), F32),
        mesh=_sc_mesh(),
        compiler_params=pltpu.CompilerParams(use_tc_tiling_on_sc=False),
        scratch_types=[
            pltpu.VMEM_SHARED((N_ACC, H), F32),
            pltpu.VMEM_SHARED((N, H), F32),
            pltpu.VMEM((BPT, IBLK), jnp.int32),
            pltpu.VMEM((BPT, IBLK), jnp.int32),
            pltpu.VMEM((PH * IBLK, H), F32),
            pltpu.VMEM((PH * IBLK, H), F32),
        ],
    )
    return f(y, src2, dst2, zeros)


NS8 = N // 8


def _tc_first(x3, W1, d0s, d1s):
    def body(x_ref, w_ref, d0_ref, d1_ref, y_ref, dinv_ref):
        dinv = lax.rsqrt(d0_ref[...] + d1_ref[...] + 1.0)
        parts = [jnp.dot(x_ref[:, j, :], w_ref[...], preferred_element_type=F32)
                 for j in range(8)]
        y_ref[...] = jnp.concatenate(parts, axis=1) * dinv
        dinv_ref[...] = dinv

    return pl.pallas_call(
        body,
        out_shape=(jax.ShapeDtypeStruct((NS8, 128), F32),
                   jax.ShapeDtypeStruct((NS8, 128), F32)),
    )(x3, W1, d0s, d1s)


def _tc_combine_mm(s0, s1, y, dinv, b, Wbd):
    def body(s0_ref, s1_ref, y_ref, dinv_ref, b_ref, w_ref, o_ref):
        dinv = dinv_ref[...]
        h = jnp.maximum(dinv * (s0_ref[...] + s1_ref[...] + y_ref[...]) + b_ref[...], 0.0)
        o_ref[...] = jnp.dot(h, w_ref[...], preferred_element_type=F32) * dinv

    return pl.pallas_call(
        body, out_shape=jax.ShapeDtypeStruct((NS8, 128), F32),
    )(s0, s1, y, dinv, b, Wbd)


def _tc_combine(s0, s1, y, dinv, b):
    def body(s0_ref, s1_ref, y_ref, dinv_ref, b_ref, o_ref):
        o_ref[...] = jnp.maximum(
            dinv_ref[...] * (s0_ref[...] + s1_ref[...] + y_ref[...]) + b_ref[...], 0.0)

    return pl.pallas_call(
        body, out_shape=jax.ShapeDtypeStruct((NS8, 128), F32),
    )(s0, s1, y, dinv, b)


MV_BK = 6400


def _tc_matvec_heads(flat, Wp1, Wv1, Wp2t, bp1, bp2, bv1, wiv, biv, wev, bev):
    grid = (N * H) // MV_BK
    wrows = N // grid

    def body(f_ref, a_ref, b_ref, wp2_ref, bp1_ref, bp2_ref, bv1_ref,
             wiv_ref, biv_ref, wev_ref, bev_ref,
             x_ref, ev_ref, iv_ref, pacc, vacc, wp2_full):
        k = pl.program_id(0)

        @pl.when(k == 0)
        def _():
            pacc[...] = jnp.zeros_like(pacc)
            vacc[...] = jnp.zeros_like(vacc)

        f = f_ref[...]
        pacc[...] += jnp.dot(f, a_ref[...], preferred_element_type=F32)
        vacc[...] += jnp.dot(f, b_ref[...], preferred_element_type=F32)
        wp2_full[pl.ds(k * wrows, wrows), :] = wp2_ref[...]

        @pl.when(k == grid - 1)
        def _():
            ph = jnp.maximum(pacc[...] + bp1_ref[...], 0.0)
            x_ref[...] = lax.dot_general(
                ph, wp2_full[...], (((1,), (1,)), ((), ())),
                preferred_element_type=F32) + bp2_ref[...]
            V = jnp.maximum(vacc[...] + bv1_ref[...], 0.0)
            iv_ref[...] = jnp.sum(V * wiv_ref[...], axis=1, keepdims=True) + biv_ref[...]
            ev_ref[...] = jnp.sum(V * wev_ref[...], axis=1, keepdims=True) + bev_ref[...]

    c0 = lambda k: (0, 0)
    return pl.pallas_call(
        body,
        grid=(grid,),
        in_specs=[
            pl.BlockSpec((1, MV_BK), lambda k: (0, k)),
            pl.BlockSpec((MV_BK, 256), lambda k: (k, 0)),
            pl.BlockSpec((MV_BK, 256), lambda k: (k, 0)),
            pl.BlockSpec((wrows, 256), lambda k: (k, 0)),
            pl.BlockSpec((1, 256), c0),
            pl.BlockSpec((1, N), c0),
            pl.BlockSpec((1, 256), c0),
            pl.BlockSpec((1, 256), c0),
            pl.BlockSpec((1, 1), c0),
            pl.BlockSpec((1, 256), c0),
            pl.BlockSpec((1, 1), c0),
        ],
        out_specs=(pl.BlockSpec((1, N), c0),
                   pl.BlockSpec((1, 1), c0),
                   pl.BlockSpec((1, 1), c0)),
        out_shape=(jax.ShapeDtypeStruct((1, N), F32),
                   jax.ShapeDtypeStruct((1, 1), F32),
                   jax.ShapeDtypeStruct((1, 1), F32)),
        scratch_shapes=[
            pltpu.VMEM((1, 256), F32),
            pltpu.VMEM((1, 256), F32),
            pltpu.VMEM((N, 256), F32),
        ],
        compiler_params=pltpu.CompilerParams(vmem_limit_bytes=63 * 1024 * 1024),
    )(flat, Wp1, Wv1, Wp2t, bp1, bp2, bv1, wiv, biv, wev, bev)


def kernel(x, edge_index, W1, b1, W2, b2, W3, b3, Wp1, bp1, Wp2, bp2,
           Wv1, bv1, Wiv, biv, Wev, bev):
    ei = edge_index.astype(jnp.int32)
    pad = EPAD - E
    src2 = jnp.concatenate([ei[0], jnp.zeros((pad,), jnp.int32)]).reshape(EBP, IBLK)
    dpad = N + (jnp.arange(pad, dtype=jnp.int32) % (N_ACC - N))
    dst2 = jnp.concatenate([ei[1], dpad]).reshape(EBP, IBLK)
    zeros = jnp.zeros((N_ACC, H), F32)

    eye8 = jnp.eye(8, dtype=F32)

    def tile8(b):
        return jnp.tile(b.reshape(1, H), (1, 8))

    degp = _sc_degree(dst2, zeros)
    d0s = degp[0].reshape(NS8, 128)
    d1s = degp[1].reshape(NS8, 128)
    y1s, dinvs = _tc_first(x.reshape(NS8, 8, 128), W1, d0s, d1s)

    p1 = _sc_gather_scatter(y1s.reshape(N, H), src2, dst2, zeros)
    y2s = _tc_combine_mm(p1[0].reshape(NS8, 128), p1[1].reshape(NS8, 128),
                         y1s, dinvs, tile8(b1), jnp.kron(eye8, W2))

    p2 = _sc_gather_scatter(y2s.reshape(N, H), src2, dst2, zeros)
    y3s = _tc_combine_mm(p2[0].reshape(NS8, 128), p2[1].reshape(NS8, 128),
                         y2s, dinvs, tile8(b2), jnp.kron(eye8, W3))

    p3 = _sc_gather_scatter(y3s.reshape(N, H), src2, dst2, zeros)
    h3s = _tc_combine(p3[0].reshape(NS8, 128), p3[1].reshape(NS8, 128),
                      y3s, dinvs, tile8(b3))

    flat = h3s.reshape(1, N * H)
    X, eV, iV = _tc_matvec_heads(
        flat, Wp1, Wv1, Wp2.T, bp1.reshape(1, 256), bp2.reshape(1, N),
        bv1.reshape(1, 256), Wiv.reshape(1, 256), biv.reshape(1, 1),
        Wev.reshape(1, 256), bev.reshape(1, 1))
    return (X, eV, iV)
